# baseline (speedup 1.0000x reference)
import functools

import jax
import jax.numpy as jnp
from jax import lax
from jax.experimental import pallas as pl
from jax.experimental.pallas import tpu as pltpu
from jax.experimental.pallas import tpu_sc as plsc

N = 10000
E = 320000
H = 128

NC = 2
NS = 16
EPW = E // (NC * NS)
CHUNK = 128
NCHUNK = -(-EPW // CHUNK)
EPWP = NCHUNK * CHUNK
ACCR = N + 8
ZROWS = 80
NZCH = N // ZROWS
NZIT = -(-NZCH // NS)

_SQRT_HALF = 0.7071067811865476


def _gelu(x):
    return 0.5 * x * (1.0 + lax.erf(x * _SQRT_HALF))


def _bn(x, g, b):
    mu = jnp.mean(x, axis=0, keepdims=True)
    xc = x - mu
    var = jnp.mean(xc * xc, axis=0, keepdims=True)
    return xc * lax.rsqrt(var + 1e-5) * g + b


def _mm(a, b):
    return jnp.dot(a, b, precision=lax.Precision.HIGHEST,
                   preferred_element_type=jnp.float32)


def _tc1_body(x, bn0g, bn0b, w0, b0, bn1g, bn1b, ws1, bc1, n1_o, t1_o):
    h = _gelu(_mm(_bn(x[...], bn0g[...], bn0b[...]), w0[...]) + b0[...])
    n1 = _bn(h, bn1g[...], bn1b[...])
    n1_o[...] = n1
    t1_o[...] = _mm(n1, ws1[...]) + n1 + bc1[...]


def _tc2_body(t1, sums, cntp, wn1, bn2g, bn2b, ws2, bc2, n2_o, t2_o):
    s = sums[0] + sums[1]
    cnt = cntp[0, :, 0:1] + cntp[1, :, 0:1]
    hn = s * (1.0 / jnp.maximum(cnt, 1.0))
    h = _gelu(t1[...] + _mm(hn, wn1[...]))
    n2 = _bn(h, bn2g[...], bn2b[...])
    n2_o[...] = n2
    t2_o[...] = _mm(n2, ws2[...]) + n2 + bc2[...]


def _tc3_body(t2, sums, cntp, wn2, bnpg, bnpb, wp, bp, out_o):
    s = sums[0] + sums[1]
    cnt = cntp[0, :, 0:1] + cntp[1, :, 0:1]
    hn = s * (1.0 / jnp.maximum(cnt, 1.0))
    h = _gelu(t2[...] + _mm(hn, wn2[...]))
    out_o[...] = _mm(_bn(h, bnpg[...], bnpb[...]), wp[...]) + bp[...]


_f32 = jnp.float32


def _tc1(x, bn0g, bn0b, w0, b0, bn1g, bn1b, ws1, bc1):
    return pl.pallas_call(
        _tc1_body,
        out_shape=(jax.ShapeDtypeStruct((N, H), _f32),
                   jax.ShapeDtypeStruct((N, H), _f32)),
    )(x, bn0g, bn0b, w0, b0, bn1g, bn1b, ws1, bc1)


def _tc2(t1, sums, cntp, wn1, bn2g, bn2b, ws2, bc2):
    return pl.pallas_call(
        _tc2_body,
        out_shape=(jax.ShapeDtypeStruct((N, H), _f32),
                   jax.ShapeDtypeStruct((N, H), _f32)),
    )(t1, sums, cntp, wn1, bn2g, bn2b, ws2, bc2)


def _tc3(t2, sums, cntp, wn2, bnpg, bnpb, wp, bp):
    return pl.pallas_call(
        _tc3_body,
        out_shape=jax.ShapeDtypeStruct((N, 1), _f32),
    )(t2, sums, cntp, wn2, bnpg, bnpb, wp, bp)


def _zero_rows(buf, nrows, ncol16):
    def body(i, _):
        for j in range(ncol16):
            buf[i, pl.ds(j * 16, 16)] = jnp.zeros((16,), _f32)
        return 0
    lax.fori_loop(0, nrows, body, 0)


def _sc_body(src_h, dst_h, tab_h, out_s, acc_sh, rows0, rows1,
             src_v, dst0, dst1, gsem, isem):
    cid = lax.axis_index("c")
    sid = lax.axis_index("s")
    wid = cid * NS + sid

    pltpu.sync_copy(src_h.at[pl.ds(wid * EPWP, EPWP)], src_v)

    _zero_rows(rows0, ZROWS, H // 16)
    for k in range(NZIT):
        cidx = sid + NS * k

        @pl.when(cidx < NZCH)
        def _():
            pltpu.sync_copy(rows0.at[pl.ds(0, ZROWS)],
                            acc_sh.at[pl.ds(cidx * ZROWS, ZROWS)])
    plsc.subcore_barrier()

    rows = (rows0, rows1)
    dsts = (dst0, dst1)
    base = wid * EPWP

    def issue(c, b):
        pltpu.async_copy(tab_h.at[src_v.at[pl.ds(c * CHUNK, CHUNK)]],
                         rows[b], gsem)
        pltpu.async_copy(dst_h.at[pl.ds(base + c * CHUNK, CHUNK)],
                         dsts[b], isem)

    issue(0, 0)

    def pair_body(i, _):
        o = i * 2
        for b in range(2):
            c = o + b
            pltpu.make_async_copy(tab_h.at[pl.ds(0, CHUNK)], rows[b],
                                  gsem).wait()
            pltpu.make_async_copy(dst_h.at[pl.ds(0, CHUNK)], dsts[b],
                                  isem).wait()
            issue(c + 1, 1 - b)
            pltpu.sync_copy(rows[b], acc_sh.at[dsts[b]], add=True)
        return 0

    lax.fori_loop(0, (NCHUNK - 1) // 2, pair_body, 0)
    pltpu.make_async_copy(tab_h.at[pl.ds(0, CHUNK)], rows0, gsem).wait()
    pltpu.make_async_copy(dst_h.at[pl.ds(0, CHUNK)], dst0, isem).wait()
    pltpu.sync_copy(rows0, acc_sh.at[dst0], add=True)
    plsc.subcore_barrier()

    for k in range(NZIT):
        cidx = sid + NS * k

        @pl.when(cidx < NZCH)
        def _():
            rr = cidx * ZROWS
            pltpu.sync_copy(acc_sh.at[pl.ds(rr, ZROWS)],
                            rows1.at[pl.ds(0, ZROWS)])
            pltpu.sync_copy(rows1.at[pl.ds(0, ZROWS)],
                            out_s.at[cid, pl.ds(rr, ZROWS)])


def _sc_cnt_body(dst_h, out_c, cnt_sh, zbuf, ones_v, dst0, dst1, isem):
    cid = lax.axis_index("c")
    sid = lax.axis_index("s")
    base = (cid * NS + sid) * EPWP
    _zero_rows(zbuf, ZROWS, H // 16)

    def fill_ones(i, _):
        for j in range(H // 16):
            ones_v[i, pl.ds(j * 16, 16)] = jnp.ones((16,), _f32)
        return 0
    lax.fori_loop(0, CHUNK, fill_ones, 0)

    for k in range(NZIT):
        cidx = sid + NS * k

        @pl.when(cidx < NZCH)
        def _():
            pltpu.sync_copy(zbuf, cnt_sh.at[pl.ds(cidx * ZROWS, ZROWS)])
    plsc.subcore_barrier()

    dsts = (dst0, dst1)
    pltpu.async_copy(dst_h.at[pl.ds(base, CHUNK)], dst0, isem)

    def pair_body(i, _):
        o = i * 2
        for b in range(2):
            c = o + b
            pltpu.make_async_copy(dst_h.at[pl.ds(0, CHUNK)], dsts[b],
                                  isem).wait()
            pltpu.async_copy(dst_h.at[pl.ds(base + (c + 1) * CHUNK, CHUNK)],
                             dsts[1 - b], isem)
            pltpu.sync_copy(ones_v, cnt_sh.at[dsts[b]], add=True)
        return 0

    lax.fori_loop(0, (NCHUNK - 1) // 2, pair_body, 0)
    pltpu.make_async_copy(dst_h.at[pl.ds(0, CHUNK)], dst0, isem).wait()
    pltpu.sync_copy(ones_v, cnt_sh.at[dst0], add=True)
    plsc.subcore_barrier()

    for k in range(NZIT):
        cidx = sid + NS * k

        @pl.when(cidx < NZCH)
        def _():
            rr = cidx * ZROWS
            pltpu.sync_copy(cnt_sh.at[pl.ds(rr, ZROWS)], zbuf)
            pltpu.sync_copy(zbuf, out_c.at[cid, pl.ds(rr, ZROWS)])


def _sc_mesh():
    return plsc.VectorSubcoreMesh(core_axis_name="c", subcore_axis_name="s",
                                  num_cores=NC, num_subcores=NS)


def _sc_agg(src, dst, table):
    return pl.kernel(
        _sc_body,
        out_type=jax.ShapeDtypeStruct((NC, N, H), _f32),
        mesh=_sc_mesh(),
        scratch_types=[
            pltpu.VMEM_SHARED((ACCR, H), _f32),
            pltpu.VMEM((CHUNK, H), _f32),
            pltpu.VMEM((CHUNK, H), _f32),
            pltpu.VMEM((EPWP,), jnp.int32),
            pltpu.VMEM((CHUNK,), jnp.int32),
            pltpu.VMEM((CHUNK,), jnp.int32),
            pltpu.SemaphoreType.DMA,
            pltpu.SemaphoreType.DMA,
        ],
    )(src, dst, table)


def _sc_cnt(dst):
    return pl.kernel(
        _sc_cnt_body,
        out_type=jax.ShapeDtypeStruct((NC, N, H), _f32),
        mesh=_sc_mesh(),
        scratch_types=[
            pltpu.VMEM_SHARED((ACCR, H), _f32),
            pltpu.VMEM((ZROWS, H), _f32),
            pltpu.VMEM((CHUNK, H), _f32),
            pltpu.VMEM((CHUNK,), jnp.int32),
            pltpu.VMEM((CHUNK,), jnp.int32),
            pltpu.SemaphoreType.DMA,
        ],
    )(dst)


def kernel(x, edge_index, bn0_g, bn0_b, W0, b0, bn1_g, bn1_b, Ws1, Wn1, bc1,
           bn2_g, bn2_b, Ws2, Wn2, bc2, bnp_g, bnp_b, Wp, bp):
    row = lambda v: v.reshape(1, -1).astype(_f32)
    pad = EPWP - EPW
    src = jnp.pad(edge_index[0].reshape(NC * NS, EPW),
                  ((0, 0), (0, pad))).reshape(-1)
    dst = jnp.pad(edge_index[1].reshape(NC * NS, EPW),
                  ((0, 0), (0, pad)), constant_values=N).reshape(-1)

    n1, t1 = _tc1(x, row(bn0_g), row(bn0_b), W0, row(b0),
                  row(bn1_g), row(bn1_b), Ws1, row(bc1))
    cntp = _sc_cnt(dst)
    sums1 = _sc_agg(src, dst, n1)
    n2, t2 = _tc2(t1, sums1, cntp, Wn1, row(bn2_g), row(bn2_b), Ws2, row(bc2))
    sums2 = _sc_agg(src, dst, n2)
    out = _tc3(t2, sums2, cntp, Wn2, row(bnp_g), row(bnp_b), Wp, row(bp))
    return out


# speedup vs baseline: 1.3938x; 1.3938x over previous
import functools

import jax
import jax.numpy as jnp
from jax import lax
from jax.experimental import pallas as pl
from jax.experimental.pallas import tpu as pltpu
from jax.experimental.pallas import tpu_sc as plsc

N = 10000
E = 320000
H = 128

NC = 2
NS = 16
EPW = E // (NC * NS)
CHUNK = 80
NCHUNK = EPW // CHUNK
EPWP = EPW
ACCR = N
ZROWS = 40
NZCH = N // ZROWS
NZIT = -(-NZCH // NS)

_SQRT_HALF = 0.7071067811865476


def _gelu(x):
    return 0.5 * x * (1.0 + lax.erf(x * _SQRT_HALF))


def _bn(x, g, b):
    mu = jnp.mean(x, axis=0, keepdims=True)
    xc = x - mu
    var = jnp.mean(xc * xc, axis=0, keepdims=True)
    return xc * lax.rsqrt(var + 1e-5) * g + b


def _mm(a, b):
    return jnp.dot(a, b, precision=lax.Precision.HIGHEST,
                   preferred_element_type=jnp.float32)


def _tc1_body(x, bn0g, bn0b, w0, b0, bn1g, bn1b, ws1, bc1, n1_o, t1_o):
    h = _gelu(_mm(_bn(x[...], bn0g[...], bn0b[...]), w0[...]) + b0[...])
    n1 = _bn(h, bn1g[...], bn1b[...])
    n1_o[...] = n1
    t1_o[...] = _mm(n1, ws1[...]) + n1 + bc1[...]


def _tc2_body(t1, sums, cntp, wn1, bn2g, bn2b, ws2, bc2, n2_o, t2_o):
    s = sums[0] + sums[1]
    cnt = cntp[0, :, 0:1] + cntp[1, :, 0:1]
    hn = s * (1.0 / jnp.maximum(cnt, 1.0))
    h = _gelu(t1[...] + _mm(hn, wn1[...]))
    n2 = _bn(h, bn2g[...], bn2b[...])
    n2_o[...] = n2
    t2_o[...] = _mm(n2, ws2[...]) + n2 + bc2[...]


def _tc3_body(t2, sums, cntp, wn2, bnpg, bnpb, wp, bp, out_o):
    s = sums[0] + sums[1]
    cnt = cntp[0, :, 0:1] + cntp[1, :, 0:1]
    hn = s * (1.0 / jnp.maximum(cnt, 1.0))
    h = _gelu(t2[...] + _mm(hn, wn2[...]))
    out_o[...] = _mm(_bn(h, bnpg[...], bnpb[...]), wp[...]) + bp[...]


_f32 = jnp.float32


def _tc1(x, bn0g, bn0b, w0, b0, bn1g, bn1b, ws1, bc1):
    return pl.pallas_call(
        _tc1_body,
        out_shape=(jax.ShapeDtypeStruct((N, H), _f32),
                   jax.ShapeDtypeStruct((N, H), _f32)),
    )(x, bn0g, bn0b, w0, b0, bn1g, bn1b, ws1, bc1)


def _tc2(t1, sums, cntp, wn1, bn2g, bn2b, ws2, bc2):
    return pl.pallas_call(
        _tc2_body,
        out_shape=(jax.ShapeDtypeStruct((N, H), _f32),
                   jax.ShapeDtypeStruct((N, H), _f32)),
    )(t1, sums, cntp, wn1, bn2g, bn2b, ws2, bc2)


def _tc3(t2, sums, cntp, wn2, bnpg, bnpb, wp, bp):
    return pl.pallas_call(
        _tc3_body,
        out_shape=jax.ShapeDtypeStruct((N, 1), _f32),
    )(t2, sums, cntp, wn2, bnpg, bnpb, wp, bp)


def _zero_rows(buf, nrows, ncol16):
    def body(i, _):
        for j in range(ncol16):
            buf[i, pl.ds(j * 16, 16)] = jnp.zeros((16,), _f32)
        return 0
    lax.fori_loop(0, nrows, body, 0)


def _sc_body(src_h, dst_h, tab_h, out_s, acc_sh, rows0, rows1,
             src_v, dst0, dst1, gsem, isem):
    cid = lax.axis_index("c")
    sid = lax.axis_index("s")
    wid = cid * NS + sid

    pltpu.sync_copy(src_h.at[pl.ds(wid * EPWP, EPWP)], src_v)

    _zero_rows(rows0, ZROWS, H // 16)
    for k in range(NZIT):
        cidx = sid + NS * k

        @pl.when(cidx < NZCH)
        def _():
            pltpu.sync_copy(rows0.at[pl.ds(0, ZROWS)],
                            acc_sh.at[pl.ds(cidx * ZROWS, ZROWS)])
    plsc.subcore_barrier()

    rows = (rows0, rows1)
    dsts = (dst0, dst1)
    base = wid * EPWP

    def issue(c, b):
        pltpu.async_copy(tab_h.at[src_v.at[pl.ds(c * CHUNK, CHUNK)]],
                         rows[b], gsem)
        pltpu.async_copy(dst_h.at[pl.ds(base + c * CHUNK, CHUNK)],
                         dsts[b], isem)

    issue(0, 0)

    def pair_body(i, _):
        o = i * 2
        for b in range(2):
            c = o + b
            pltpu.make_async_copy(tab_h.at[pl.ds(0, CHUNK)], rows[b],
                                  gsem).wait()
            pltpu.make_async_copy(dst_h.at[pl.ds(0, CHUNK)], dsts[b],
                                  isem).wait()
            issue(c + 1, 1 - b)
            pltpu.sync_copy(rows[b], acc_sh.at[dsts[b]], add=True)
        return 0

    lax.fori_loop(0, (NCHUNK - 1) // 2, pair_body, 0)
    pltpu.make_async_copy(tab_h.at[pl.ds(0, CHUNK)], rows0, gsem).wait()
    pltpu.make_async_copy(dst_h.at[pl.ds(0, CHUNK)], dst0, isem).wait()
    pltpu.sync_copy(rows0, acc_sh.at[dst0], add=True)
    plsc.subcore_barrier()

    for k in range(NZIT):
        cidx = sid + NS * k

        @pl.when(cidx < NZCH)
        def _():
            rr = cidx * ZROWS
            pltpu.sync_copy(acc_sh.at[pl.ds(rr, ZROWS)],
                            rows1.at[pl.ds(0, ZROWS)])
            pltpu.sync_copy(rows1.at[pl.ds(0, ZROWS)],
                            out_s.at[cid, pl.ds(rr, ZROWS)])


def _sc_cnt_body(dst_h, out_c, cnt_sh, zbuf, ones_v, dst0, dst1, isem):
    cid = lax.axis_index("c")
    sid = lax.axis_index("s")
    base = (cid * NS + sid) * EPWP
    _zero_rows(zbuf, ZROWS, H // 16)

    def fill_ones(i, _):
        for j in range(H // 16):
            ones_v[i, pl.ds(j * 16, 16)] = jnp.ones((16,), _f32)
        return 0
    lax.fori_loop(0, CHUNK, fill_ones, 0)

    for k in range(NZIT):
        cidx = sid + NS * k

        @pl.when(cidx < NZCH)
        def _():
            pltpu.sync_copy(zbuf, cnt_sh.at[pl.ds(cidx * ZROWS, ZROWS)])
    plsc.subcore_barrier()

    dsts = (dst0, dst1)
    pltpu.async_copy(dst_h.at[pl.ds(base, CHUNK)], dst0, isem)

    def pair_body(i, _):
        o = i * 2
        for b in range(2):
            c = o + b
            pltpu.make_async_copy(dst_h.at[pl.ds(0, CHUNK)], dsts[b],
                                  isem).wait()
            pltpu.async_copy(dst_h.at[pl.ds(base + (c + 1) * CHUNK, CHUNK)],
                             dsts[1 - b], isem)
            pltpu.sync_copy(ones_v, cnt_sh.at[dsts[b]], add=True)
        return 0

    lax.fori_loop(0, (NCHUNK - 1) // 2, pair_body, 0)
    pltpu.make_async_copy(dst_h.at[pl.ds(0, CHUNK)], dst0, isem).wait()
    pltpu.sync_copy(ones_v, cnt_sh.at[dst0], add=True)
    plsc.subcore_barrier()

    for k in range(NZIT):
        cidx = sid + NS * k

        @pl.when(cidx < NZCH)
        def _():
            rr = cidx * ZROWS
            pltpu.sync_copy(cnt_sh.at[pl.ds(rr, ZROWS)], zbuf)
            pltpu.sync_copy(zbuf, out_c.at[cid, pl.ds(rr, ZROWS)])


def _sc_mesh():
    return plsc.VectorSubcoreMesh(core_axis_name="c", subcore_axis_name="s",
                                  num_cores=NC, num_subcores=NS)


def _sc_agg(src, dst, table):
    return pl.kernel(
        _sc_body,
        out_type=jax.ShapeDtypeStruct((NC, N, H), _f32),
        mesh=_sc_mesh(),
        scratch_types=[
            pltpu.VMEM_SHARED((ACCR, H), _f32),
            pltpu.VMEM((CHUNK, H), _f32),
            pltpu.VMEM((CHUNK, H), _f32),
            pltpu.VMEM((EPWP,), jnp.int32),
            pltpu.VMEM((CHUNK,), jnp.int32),
            pltpu.VMEM((CHUNK,), jnp.int32),
            pltpu.SemaphoreType.DMA,
            pltpu.SemaphoreType.DMA,
        ],
    )(src, dst, table)


def _sc_cnt(dst):
    return pl.kernel(
        _sc_cnt_body,
        out_type=jax.ShapeDtypeStruct((NC, N, H), _f32),
        mesh=_sc_mesh(),
        scratch_types=[
            pltpu.VMEM_SHARED((ACCR, H), _f32),
            pltpu.VMEM((ZROWS, H), _f32),
            pltpu.VMEM((CHUNK, H), _f32),
            pltpu.VMEM((CHUNK,), jnp.int32),
            pltpu.VMEM((CHUNK,), jnp.int32),
            pltpu.SemaphoreType.DMA,
        ],
    )(dst)


def kernel(x, edge_index, bn0_g, bn0_b, W0, b0, bn1_g, bn1_b, Ws1, Wn1, bc1,
           bn2_g, bn2_b, Ws2, Wn2, bc2, bnp_g, bnp_b, Wp, bp):
    row = lambda v: v.reshape(1, -1).astype(_f32)
    src = edge_index[0]
    dst = edge_index[1]

    n1, t1 = _tc1(x, row(bn0_g), row(bn0_b), W0, row(b0),
                  row(bn1_g), row(bn1_b), Ws1, row(bc1))
    cntp = _sc_cnt(dst)
    sums1 = _sc_agg(src, dst, n1)
    n2, t2 = _tc2(t1, sums1, cntp, Wn1, row(bn2_g), row(bn2_b), Ws2, row(bc2))
    sums2 = _sc_agg(src, dst, n2)
    out = _tc3(t2, sums2, cntp, Wn2, row(bnp_g), row(bnp_b), Wp, row(bp))
    return out


# speedup vs baseline: 1.4015x; 1.0055x over previous
import functools

import jax
import jax.numpy as jnp
from jax import lax
from jax.experimental import pallas as pl
from jax.experimental.pallas import tpu as pltpu
from jax.experimental.pallas import tpu_sc as plsc

N = 10000
E = 320000
H = 128

NC = 2
NS = 16
EPW = E // (NC * NS)
CHUNK = 80
NCHUNK = EPW // CHUNK
EPWP = EPW
ACCR = N
ZROWS = 40
NZCH = N // ZROWS
NZIT = -(-NZCH // NS)

_SQRT_HALF = 0.7071067811865476


def _gelu(x):
    return 0.5 * x * (1.0 + lax.erf(x * _SQRT_HALF))


def _bn(x, g, b):
    mu = jnp.mean(x, axis=0, keepdims=True)
    xc = x - mu
    var = jnp.mean(xc * xc, axis=0, keepdims=True)
    return xc * lax.rsqrt(var + 1e-5) * g + b


def _mm(a, b):
    return jnp.dot(a, b, precision=lax.Precision.HIGHEST,
                   preferred_element_type=jnp.float32)


def _tc1_body(x, bn0g, bn0b, w0, b0, bn1g, bn1b, ws1, bc1, n1_o, t1_o):
    h = _gelu(_mm(_bn(x[...], bn0g[...], bn0b[...]), w0[...]) + b0[...])
    n1 = _bn(h, bn1g[...], bn1b[...])
    n1_o[...] = n1
    t1_o[...] = _mm(n1, ws1[...]) + n1 + bc1[...]


def _tc2_body(t1, sums, cntp, wn1, bn2g, bn2b, ws2, bc2, n2_o, t2_o):
    s = sums[0] + sums[1]
    cnt = cntp[0, :, 0:1] + cntp[1, :, 0:1]
    hn = s * (1.0 / jnp.maximum(cnt, 1.0))
    h = _gelu(t1[...] + _mm(hn, wn1[...]))
    n2 = _bn(h, bn2g[...], bn2b[...])
    n2_o[...] = n2
    t2_o[...] = _mm(n2, ws2[...]) + n2 + bc2[...]


def _tc3_body(t2, sums, cntp, wn2, bnpg, bnpb, wp, bp, out_o):
    s = sums[0] + sums[1]
    cnt = cntp[0, :, 0:1] + cntp[1, :, 0:1]
    hn = s * (1.0 / jnp.maximum(cnt, 1.0))
    h = _gelu(t2[...] + _mm(hn, wn2[...]))
    out_o[...] = _mm(_bn(h, bnpg[...], bnpb[...]), wp[...]) + bp[...]


_f32 = jnp.float32


def _tc1(x, bn0g, bn0b, w0, b0, bn1g, bn1b, ws1, bc1):
    return pl.pallas_call(
        _tc1_body,
        out_shape=(jax.ShapeDtypeStruct((N, H), _f32),
                   jax.ShapeDtypeStruct((N, H), _f32)),
    )(x, bn0g, bn0b, w0, b0, bn1g, bn1b, ws1, bc1)


def _tc2(t1, sums, cntp, wn1, bn2g, bn2b, ws2, bc2):
    return pl.pallas_call(
        _tc2_body,
        out_shape=(jax.ShapeDtypeStruct((N, H), _f32),
                   jax.ShapeDtypeStruct((N, H), _f32)),
    )(t1, sums, cntp, wn1, bn2g, bn2b, ws2, bc2)


def _tc3(t2, sums, cntp, wn2, bnpg, bnpb, wp, bp):
    return pl.pallas_call(
        _tc3_body,
        out_shape=jax.ShapeDtypeStruct((N, 1), _f32),
    )(t2, sums, cntp, wn2, bnpg, bnpb, wp, bp)


def _zero_rows(buf, nrows, ncol16):
    def body(i, _):
        for j in range(ncol16):
            buf[i, pl.ds(j * 16, 16)] = jnp.zeros((16,), _f32)
        return 0
    lax.fori_loop(0, nrows, body, 0)


def _sc_body(src_h, dst_h, tab_h, out_s, acc_sh, rows0, rows1, rows2,
             src_v, dst0, dst1, dst2, gsem, isem, ssem):
    cid = lax.axis_index("c")
    sid = lax.axis_index("s")
    wid = cid * NS + sid

    pltpu.sync_copy(src_h.at[pl.ds(wid * EPW, EPW)], src_v)

    _zero_rows(rows0, ZROWS, H // 16)
    for k in range(NZIT):
        cidx = sid + NS * k

        @pl.when(cidx < NZCH)
        def _():
            pltpu.sync_copy(rows0.at[pl.ds(0, ZROWS)],
                            acc_sh.at[pl.ds(cidx * ZROWS, ZROWS)])
    plsc.subcore_barrier()

    rows = (rows0, rows1, rows2)
    dsts = (dst0, dst1, dst2)
    base = wid * EPW

    def issue(c, b):
        pltpu.async_copy(tab_h.at[src_v.at[pl.ds(c * CHUNK, CHUNK)]],
                         rows[b], gsem)
        pltpu.async_copy(dst_h.at[pl.ds(base + c * CHUNK, CHUNK)],
                         dsts[b], isem)

    def wait_in(b):
        pltpu.make_async_copy(tab_h.at[pl.ds(0, CHUNK)], rows[b], gsem).wait()
        pltpu.make_async_copy(dst_h.at[pl.ds(0, CHUNK)], dsts[b], isem).wait()

    def scat(b):
        pltpu.async_copy(rows[b], acc_sh.at[dsts[b]], ssem, add=True)

    def wait_s(b):
        pltpu.make_async_copy(rows[b], acc_sh.at[dsts[b]], ssem).wait()

    issue(0, 0)
    wait_in(0)
    issue(1, 1)
    scat(0)

    def tri_body(i, _):
        o = 3 * i + 1
        for j in range(3):
            c = o + j
            b = (1 + j) % 3
            wait_in(b)
            wait_s(j % 3)
            issue(c + 1, (2 + j) % 3)
            scat(b)
        return 0

    lax.fori_loop(0, (NCHUNK - 2) // 3, tri_body, 0)
    wait_in((NCHUNK - 1) % 3)
    wait_s((NCHUNK - 2) % 3)
    scat((NCHUNK - 1) % 3)
    wait_s((NCHUNK - 1) % 3)
    plsc.subcore_barrier()

    for k in range(NZIT):
        cidx = sid + NS * k

        @pl.when(cidx < NZCH)
        def _():
            rr = cidx * ZROWS
            pltpu.sync_copy(acc_sh.at[pl.ds(rr, ZROWS)],
                            rows1.at[pl.ds(0, ZROWS)])
            pltpu.sync_copy(rows1.at[pl.ds(0, ZROWS)],
                            out_s.at[cid, pl.ds(rr, ZROWS)])


def _sc_cnt_body(dst_h, out_c, cnt_sh, zbuf, ones_v, dst0, dst1, dst2,
                 isem, ssem):
    cid = lax.axis_index("c")
    sid = lax.axis_index("s")
    base = (cid * NS + sid) * EPW
    _zero_rows(zbuf, ZROWS, H // 16)

    def fill_ones(i, _):
        for j in range(H // 16):
            ones_v[i, pl.ds(j * 16, 16)] = jnp.ones((16,), _f32)
        return 0
    lax.fori_loop(0, CHUNK, fill_ones, 0)

    for k in range(NZIT):
        cidx = sid + NS * k

        @pl.when(cidx < NZCH)
        def _():
            pltpu.sync_copy(zbuf, cnt_sh.at[pl.ds(cidx * ZROWS, ZROWS)])
    plsc.subcore_barrier()

    dsts = (dst0, dst1, dst2)

    def issue(c, b):
        pltpu.async_copy(dst_h.at[pl.ds(base + c * CHUNK, CHUNK)],
                         dsts[b], isem)

    def wait_in(b):
        pltpu.make_async_copy(dst_h.at[pl.ds(0, CHUNK)], dsts[b], isem).wait()

    def scat(b):
        pltpu.async_copy(ones_v, cnt_sh.at[dsts[b]], ssem, add=True)

    def wait_s(b):
        pltpu.make_async_copy(ones_v, cnt_sh.at[dsts[b]], ssem).wait()

    issue(0, 0)
    wait_in(0)
    issue(1, 1)
    scat(0)

    def tri_body(i, _):
        o = 3 * i + 1
        for j in range(3):
            c = o + j
            b = (1 + j) % 3
            wait_in(b)
            wait_s(j % 3)
            issue(c + 1, (2 + j) % 3)
            scat(b)
        return 0

    lax.fori_loop(0, (NCHUNK - 2) // 3, tri_body, 0)
    wait_in((NCHUNK - 1) % 3)
    wait_s((NCHUNK - 2) % 3)
    scat((NCHUNK - 1) % 3)
    wait_s((NCHUNK - 1) % 3)
    plsc.subcore_barrier()

    for k in range(NZIT):
        cidx = sid + NS * k

        @pl.when(cidx < NZCH)
        def _():
            rr = cidx * ZROWS
            pltpu.sync_copy(cnt_sh.at[pl.ds(rr, ZROWS)], zbuf)
            pltpu.sync_copy(zbuf, out_c.at[cid, pl.ds(rr, ZROWS)])


def _sc_mesh():
    return plsc.VectorSubcoreMesh(core_axis_name="c", subcore_axis_name="s",
                                  num_cores=NC, num_subcores=NS)


def _sc_agg(src, dst, table):
    return pl.kernel(
        _sc_body,
        out_type=jax.ShapeDtypeStruct((NC, N, H), _f32),
        mesh=_sc_mesh(),
        scratch_types=[
            pltpu.VMEM_SHARED((ACCR, H), _f32),
            pltpu.VMEM((CHUNK, H), _f32),
            pltpu.VMEM((CHUNK, H), _f32),
            pltpu.VMEM((CHUNK, H), _f32),
            pltpu.VMEM((EPW,), jnp.int32),
            pltpu.VMEM((CHUNK,), jnp.int32),
            pltpu.VMEM((CHUNK,), jnp.int32),
            pltpu.VMEM((CHUNK,), jnp.int32),
            pltpu.SemaphoreType.DMA,
            pltpu.SemaphoreType.DMA,
            pltpu.SemaphoreType.DMA,
        ],
    )(src, dst, table)


def _sc_cnt(dst):
    return pl.kernel(
        _sc_cnt_body,
        out_type=jax.ShapeDtypeStruct((NC, N, H), _f32),
        mesh=_sc_mesh(),
        scratch_types=[
            pltpu.VMEM_SHARED((ACCR, H), _f32),
            pltpu.VMEM((ZROWS, H), _f32),
            pltpu.VMEM((CHUNK, H), _f32),
            pltpu.VMEM((CHUNK,), jnp.int32),
            pltpu.VMEM((CHUNK,), jnp.int32),
            pltpu.VMEM((CHUNK,), jnp.int32),
            pltpu.SemaphoreType.DMA,
            pltpu.SemaphoreType.DMA,
        ],
    )(dst)


def kernel(x, edge_index, bn0_g, bn0_b, W0, b0, bn1_g, bn1_b, Ws1, Wn1, bc1,
           bn2_g, bn2_b, Ws2, Wn2, bc2, bnp_g, bnp_b, Wp, bp):
    row = lambda v: v.reshape(1, -1).astype(_f32)
    src = edge_index[0]
    dst = edge_index[1]

    n1, t1 = _tc1(x, row(bn0_g), row(bn0_b), W0, row(b0),
                  row(bn1_g), row(bn1_b), Ws1, row(bc1))
    cntp = _sc_cnt(dst)
    sums1 = _sc_agg(src, dst, n1)
    n2, t2 = _tc2(t1, sums1, cntp, Wn1, row(bn2_g), row(bn2_b), Ws2, row(bc2))
    sums2 = _sc_agg(src, dst, n2)
    out = _tc3(t2, sums2, cntp, Wn2, row(bnp_g), row(bnp_b), Wp, row(bp))
    return out


# speedup vs baseline: 1.4270x; 1.0182x over previous
import functools

import jax
import jax.numpy as jnp
from jax import lax
from jax.experimental import pallas as pl
from jax.experimental.pallas import tpu as pltpu
from jax.experimental.pallas import tpu_sc as plsc

N = 10000
E = 320000
H = 128

NC = 2
NS = 16
EPW = E // (NC * NS)
CHUNK = 80
NCHUNK = EPW // CHUNK
EPWP = EPW
ACCR = N
ZROWS = 80
NZCH = N // ZROWS
NZIT = -(-NZCH // NS)

_SQRT_HALF = 0.7071067811865476


def _gelu(x):
    return 0.5 * x * (1.0 + lax.erf(x * _SQRT_HALF))


def _bn(x, g, b):
    mu = jnp.mean(x, axis=0, keepdims=True)
    xc = x - mu
    var = jnp.mean(xc * xc, axis=0, keepdims=True)
    return xc * lax.rsqrt(var + 1e-5) * g + b


def _mm(a, b):
    return jnp.dot(a, b, precision=lax.Precision.HIGHEST,
                   preferred_element_type=jnp.float32)


def _tc1_body(x, bn0g, bn0b, w0, b0, bn1g, bn1b, ws1, bc1, n1_o, t1_o):
    h = _gelu(_mm(_bn(x[...], bn0g[...], bn0b[...]), w0[...]) + b0[...])
    n1 = _bn(h, bn1g[...], bn1b[...])
    n1_o[...] = n1
    t1_o[...] = _mm(n1, ws1[...]) + n1 + bc1[...]


def _tc2_body(t1, sums, cntp, wn1, bn2g, bn2b, ws2, bc2, n2_o, t2_o):
    s = sums[0] + sums[1]
    cnt = cntp[0, :, 0:1] + cntp[1, :, 0:1]
    hn = s * (1.0 / jnp.maximum(cnt, 1.0))
    h = _gelu(t1[...] + _mm(hn, wn1[...]))
    n2 = _bn(h, bn2g[...], bn2b[...])
    n2_o[...] = n2
    t2_o[...] = _mm(n2, ws2[...]) + n2 + bc2[...]


def _tc3_body(t2, sums, cntp, wn2, bnpg, bnpb, wp, bp, out_o):
    s = sums[0] + sums[1]
    cnt = cntp[0, :, 0:1] + cntp[1, :, 0:1]
    hn = s * (1.0 / jnp.maximum(cnt, 1.0))
    h = _gelu(t2[...] + _mm(hn, wn2[...]))
    out_o[...] = _mm(_bn(h, bnpg[...], bnpb[...]), wp[...]) + bp[...]


_f32 = jnp.float32


def _tc1(x, bn0g, bn0b, w0, b0, bn1g, bn1b, ws1, bc1):
    return pl.pallas_call(
        _tc1_body,
        out_shape=(jax.ShapeDtypeStruct((N, H), _f32),
                   jax.ShapeDtypeStruct((N, H), _f32)),
    )(x, bn0g, bn0b, w0, b0, bn1g, bn1b, ws1, bc1)


def _tc2(t1, sums, cntp, wn1, bn2g, bn2b, ws2, bc2):
    return pl.pallas_call(
        _tc2_body,
        out_shape=(jax.ShapeDtypeStruct((N, H), _f32),
                   jax.ShapeDtypeStruct((N, H), _f32)),
    )(t1, sums, cntp, wn1, bn2g, bn2b, ws2, bc2)


def _tc3(t2, sums, cntp, wn2, bnpg, bnpb, wp, bp):
    return pl.pallas_call(
        _tc3_body,
        out_shape=jax.ShapeDtypeStruct((N, 1), _f32),
    )(t2, sums, cntp, wn2, bnpg, bnpb, wp, bp)


def _zero_rows(buf, nrows, ncol16):
    def body(i, _):
        for j in range(ncol16):
            buf[i, pl.ds(j * 16, 16)] = jnp.zeros((16,), _f32)
        return 0
    lax.fori_loop(0, nrows, body, 0)


def _sc_body(with_cnt, src_h, dst_h, tab_h, *rest):
    if with_cnt:
        (out_s, out_c, acc_sh, rows0, rows1, rows2,
         src_v, dst0, dst1, dst2, gsem, isem, ssem) = rest
    else:
        (out_s, acc_sh, rows0, rows1, rows2,
         src_v, dst0, dst1, dst2, gsem, isem, ssem) = rest
        out_c = None
    cid = lax.axis_index("c")
    sid = lax.axis_index("s")
    wid = cid * NS + sid

    pltpu.sync_copy(src_h.at[pl.ds(wid * EPW, EPW)], src_v)

    _zero_rows(rows0, ZROWS, H // 16)
    for k in range(NZIT):
        cidx = sid + NS * k

        @pl.when(cidx < NZCH)
        def _():
            pltpu.sync_copy(rows0.at[pl.ds(0, ZROWS)],
                            acc_sh.at[pl.ds(cidx * ZROWS, ZROWS)])
    plsc.subcore_barrier()

    rows = (rows0, rows1, rows2)
    dsts = (dst0, dst1, dst2)
    base = wid * EPW

    def issue(c, b):
        pltpu.async_copy(tab_h.at[src_v.at[pl.ds(c * CHUNK, CHUNK)]],
                         rows[b], gsem)
        pltpu.async_copy(dst_h.at[pl.ds(base + c * CHUNK, CHUNK)],
                         dsts[b], isem)

    def wait_in(b):
        pltpu.make_async_copy(tab_h.at[pl.ds(0, CHUNK)], rows[b], gsem).wait()
        pltpu.make_async_copy(dst_h.at[pl.ds(0, CHUNK)], dsts[b], isem).wait()

    def scat(b):
        pltpu.async_copy(rows[b], acc_sh.at[dsts[b]], ssem, add=True)

    def wait_s(b):
        pltpu.make_async_copy(rows[b], acc_sh.at[dsts[b]], ssem).wait()

    issue(0, 0)
    wait_in(0)
    issue(1, 1)
    scat(0)

    def tri_body(i, _):
        o = 3 * i + 1
        for j in range(3):
            c = o + j
            b = (1 + j) % 3
            wait_in(b)
            wait_s(j % 3)
            issue(c + 1, (2 + j) % 3)
            scat(b)
        return 0

    lax.fori_loop(0, (NCHUNK - 2) // 3, tri_body, 0)
    wait_in((NCHUNK - 1) % 3)
    wait_s((NCHUNK - 2) % 3)
    scat((NCHUNK - 1) % 3)
    wait_s((NCHUNK - 1) % 3)
    plsc.subcore_barrier()

    for k in range(NZIT):
        cidx = sid + NS * k

        @pl.when(cidx < NZCH)
        def _():
            rr = cidx * ZROWS
            pltpu.sync_copy(acc_sh.at[pl.ds(rr, ZROWS)],
                            rows1.at[pl.ds(0, ZROWS)])
            pltpu.sync_copy(rows1.at[pl.ds(0, ZROWS)],
                            out_s.at[cid, pl.ds(rr, ZROWS)])

    if not with_cnt:
        return

    plsc.subcore_barrier()
    _zero_rows(rows0, ZROWS, H // 16)

    def fill_ones(i, _):
        for j in range(H // 16):
            rows2[i, pl.ds(j * 16, 16)] = jnp.ones((16,), _f32)
        return 0
    lax.fori_loop(0, CHUNK, fill_ones, 0)

    for k in range(NZIT):
        cidx = sid + NS * k

        @pl.when(cidx < NZCH)
        def _():
            pltpu.sync_copy(rows0.at[pl.ds(0, ZROWS)],
                            acc_sh.at[pl.ds(cidx * ZROWS, ZROWS)])
    plsc.subcore_barrier()

    def cissue(c, b):
        pltpu.async_copy(dst_h.at[pl.ds(base + c * CHUNK, CHUNK)],
                         dsts[b], isem)

    def cwait_in(b):
        pltpu.make_async_copy(dst_h.at[pl.ds(0, CHUNK)], dsts[b], isem).wait()

    def cscat(b):
        pltpu.async_copy(rows2, acc_sh.at[dsts[b]], ssem, add=True)

    def cwait_s(b):
        pltpu.make_async_copy(rows2, acc_sh.at[dsts[b]], ssem).wait()

    cissue(0, 0)
    cwait_in(0)
    cissue(1, 1)
    cscat(0)

    def ctri_body(i, _):
        o = 3 * i + 1
        for j in range(3):
            c = o + j
            b = (1 + j) % 3
            cwait_in(b)
            cwait_s(j % 3)
            cissue(c + 1, (2 + j) % 3)
            cscat(b)
        return 0

    lax.fori_loop(0, (NCHUNK - 2) // 3, ctri_body, 0)
    cwait_in((NCHUNK - 1) % 3)
    cwait_s((NCHUNK - 2) % 3)
    cscat((NCHUNK - 1) % 3)
    cwait_s((NCHUNK - 1) % 3)
    plsc.subcore_barrier()

    for k in range(NZIT):
        cidx = sid + NS * k

        @pl.when(cidx < NZCH)
        def _():
            rr = cidx * ZROWS
            pltpu.sync_copy(acc_sh.at[pl.ds(rr, ZROWS)],
                            rows1.at[pl.ds(0, ZROWS)])
            pltpu.sync_copy(rows1.at[pl.ds(0, ZROWS)],
                            out_c.at[cid, pl.ds(rr, ZROWS)])


def _sc_mesh():
    return plsc.VectorSubcoreMesh(core_axis_name="c", subcore_axis_name="s",
                                  num_cores=NC, num_subcores=NS)


def _sc_agg(src, dst, table, with_cnt):
    outs = jax.ShapeDtypeStruct((NC, N, H), _f32)
    return pl.kernel(
        functools.partial(_sc_body, with_cnt),
        out_type=(outs, outs) if with_cnt else outs,
        mesh=_sc_mesh(),
        scratch_types=[
            pltpu.VMEM_SHARED((ACCR, H), _f32),
            pltpu.VMEM((CHUNK, H), _f32),
            pltpu.VMEM((CHUNK, H), _f32),
            pltpu.VMEM((CHUNK, H), _f32),
            pltpu.VMEM((EPW,), jnp.int32),
            pltpu.VMEM((CHUNK,), jnp.int32),
            pltpu.VMEM((CHUNK,), jnp.int32),
            pltpu.VMEM((CHUNK,), jnp.int32),
            pltpu.SemaphoreType.DMA,
            pltpu.SemaphoreType.DMA,
            pltpu.SemaphoreType.DMA,
        ],
    )(src, dst, table)


def kernel(x, edge_index, bn0_g, bn0_b, W0, b0, bn1_g, bn1_b, Ws1, Wn1, bc1,
           bn2_g, bn2_b, Ws2, Wn2, bc2, bnp_g, bnp_b, Wp, bp):
    row = lambda v: v.reshape(1, -1).astype(_f32)
    src = edge_index[0]
    dst = edge_index[1]

    n1, t1 = _tc1(x, row(bn0_g), row(bn0_b), W0, row(b0),
                  row(bn1_g), row(bn1_b), Ws1, row(bc1))
    sums1, cntp = _sc_agg(src, dst, n1, True)
    n2, t2 = _tc2(t1, sums1, cntp, Wn1, row(bn2_g), row(bn2_b), Ws2, row(bc2))
    sums2 = _sc_agg(src, dst, n2, False)
    out = _tc3(t2, sums2, cntp, Wn2, row(bnp_g), row(bnp_b), Wp, row(bp))
    return out
